# baseline (speedup 1.0000x reference)
import functools

import jax
import jax.numpy as jnp
from jax import lax
from jax.experimental import pallas as pl
from jax.experimental.pallas import tpu as pltpu
from jax.experimental.pallas import tpu_sc as plsc

D = 64
CH = 64
NBUF = 4
PF = 2
LAG = NBUF - PF
NC = 2
NS = 16
NW = NC * NS


def _sc_embed_concat(b, l):
    slab = b // NW
    nch = l * (slab // CH)
    mesh = plsc.VectorSubcoreMesh(core_axis_name="c", subcore_axis_name="s")

    @functools.partial(
        pl.kernel,
        out_type=jax.ShapeDtypeStruct((l, b, 2, D), jnp.float32),
        mesh=mesh,
        compiler_params=pltpu.CompilerParams(use_tc_tiling_on_sc=False),
        scratch_types=[
            pltpu.VMEM((l, slab), jnp.int32),
            pltpu.VMEM((NBUF, CH, D), jnp.float32),
            pltpu.VMEM((NBUF, CH, D), jnp.float32),
            pltpu.SemaphoreType.DMA,
            pltpu.SemaphoreType.DMA((NBUF,)),
            pltpu.SemaphoreType.DMA((NBUF,)),
            pltpu.SemaphoreType.DMA((NBUF,)),
            pltpu.SemaphoreType.DMA((NBUF,)),
        ],
    )
    def body(idx_hbm, other_hbm, table_hbm, out_hbm,
             idx_v, ebuf, obuf, sem_i, sem_g, sem_o, sem_we, sem_wo):
        wid = lax.axis_index("s") * NC + lax.axis_index("c")
        base = wid * slab
        pltpu.async_copy(
            idx_hbm.at[:, pl.ds(base, slab)], idx_v, sem_i).wait()

        nh = slab // CH

        def addr(j):
            return j // nh, (j % nh) * CH

        def start_load(j, s):
            li, off = addr(j)
            pltpu.make_async_copy(
                table_hbm.at[idx_v.at[li, pl.ds(off, CH)]], ebuf.at[s],
                sem_g.at[s]).start()
            pltpu.make_async_copy(
                other_hbm.at[li, pl.ds(base + off, CH)], obuf.at[s],
                sem_o.at[s]).start()

        def wait_load(j, s):
            li, off = addr(j)
            pltpu.make_async_copy(
                table_hbm.at[idx_v.at[li, pl.ds(off, CH)]], ebuf.at[s],
                sem_g.at[s]).wait()
            pltpu.make_async_copy(
                other_hbm.at[li, pl.ds(base + off, CH)], obuf.at[s],
                sem_o.at[s]).wait()

        def start_write(j, s):
            li, off = addr(j)
            pltpu.make_async_copy(
                ebuf.at[s], out_hbm.at[li, pl.ds(base + off, CH), 0],
                sem_we.at[s]).start()
            pltpu.make_async_copy(
                obuf.at[s], out_hbm.at[li, pl.ds(base + off, CH), 1],
                sem_wo.at[s]).start()

        def wait_write(s):
            pltpu.make_async_copy(
                ebuf.at[s], out_hbm.at[0, pl.ds(0, CH), 0],
                sem_we.at[s]).wait()
            pltpu.make_async_copy(
                obuf.at[s], out_hbm.at[0, pl.ds(0, CH), 1],
                sem_wo.at[s]).wait()

        def step(j, s, drain, prefetch):
            if drain:
                wait_write((s - LAG) % NBUF)
            if prefetch:
                start_load(j + PF, (s + PF) % NBUF)
            wait_load(j, s)
            start_write(j, s)

        for j in range(PF):
            start_load(j, j)

        w0 = NBUF
        w1 = w0 + ((nch - PF - w0) // NBUF) * NBUF
        for j in range(w0):
            step(j, j % NBUF, drain=j >= LAG, prefetch=j + PF < nch)

        def loop_body(k, carry):
            for bb in range(NBUF):
                step(w0 + k * NBUF + bb, bb, drain=True, prefetch=True)
            return carry

        lax.fori_loop(0, (w1 - w0) // NBUF, loop_body, 0)

        for j in range(w1, nch):
            step(j, j % NBUF, drain=j >= LAG, prefetch=j + PF < nch)
        for j in range(nch - LAG, nch):
            wait_write(j % NBUF)

    return body


def kernel(indices, other_features, table):
    b, l = indices.shape
    idx_t = jnp.transpose(indices).astype(jnp.int32)
    other_t = jnp.transpose(other_features, (1, 0, 2))
    out_t = _sc_embed_concat(b, l)(idx_t, other_t, table)
    return jnp.transpose(out_t.reshape(l, b, 2 * D), (1, 0, 2))


# speedup vs baseline: 4.2291x; 4.2291x over previous
import functools

import jax
import jax.numpy as jnp
from jax import lax
from jax.experimental import pallas as pl
from jax.experimental.pallas import tpu as pltpu
from jax.experimental.pallas import tpu_sc as plsc

D = 64
CH = 64
NBUF = 6
PF = 3
LAG = NBUF - PF
NC = 2
NS = 16
NW = NC * NS
RU = 4


def _sc_embed_concat(b, l):
    slab = b // NW
    nch = l * (slab // CH)
    mesh = plsc.VectorSubcoreMesh(core_axis_name="c", subcore_axis_name="s")

    @functools.partial(
        pl.kernel,
        out_type=jax.ShapeDtypeStruct((l, b, 2 * D), jnp.float32),
        mesh=mesh,
        scratch_types=[
            pltpu.VMEM((l, slab), jnp.int32),
            pltpu.VMEM((NBUF, CH, 2 * D), jnp.float32),
            pltpu.VMEM((NBUF, CH, D), jnp.float32),
            pltpu.SemaphoreType.DMA,
            pltpu.SemaphoreType.DMA((NBUF,)),
            pltpu.SemaphoreType.DMA((NBUF,)),
            pltpu.SemaphoreType.DMA((NBUF,)),
        ],
    )
    def body(idx_hbm, other_hbm, table_hbm, out_hbm,
             idx_v, cbuf, obuf, sem_i, sem_g, sem_o, sem_w):
        wid = lax.axis_index("s") * NC + lax.axis_index("c")
        base = pl.multiple_of(wid * slab, slab)
        pltpu.async_copy(
            idx_hbm.at[:, pl.ds(base, slab)], idx_v, sem_i).wait()

        nh = slab // CH

        def addr(j):
            return j // nh, (j % nh) * CH

        def start_load(j, s):
            li, off = addr(j)
            b0 = pl.multiple_of(base + off, CH)
            pltpu.make_async_copy(
                table_hbm.at[idx_v.at[li, pl.ds(off, CH)]], cbuf.at[s],
                sem_g.at[s]).start()
            pltpu.make_async_copy(
                other_hbm.at[li, pl.ds(b0, CH)], obuf.at[s],
                sem_o.at[s]).start()

        def wait_load(j, s):
            li, off = addr(j)
            b0 = pl.multiple_of(base + off, CH)
            pltpu.make_async_copy(
                table_hbm.at[idx_v.at[li, pl.ds(off, CH)]], cbuf.at[s],
                sem_g.at[s]).wait()
            pltpu.make_async_copy(
                other_hbm.at[li, pl.ds(b0, CH)], obuf.at[s],
                sem_o.at[s]).wait()

        def start_write(j, s):
            def rows(i, carry):
                for k in range(RU):
                    r = i * RU + k
                    for c in range(D // 16):
                        cbuf[s, r, pl.ds(D + c * 16, 16)] = (
                            obuf[s, r, pl.ds(c * 16, 16)])
                return carry

            lax.fori_loop(0, CH // RU, rows, 0)
            li, off = addr(j)
            b0 = pl.multiple_of(base + off, CH)
            pltpu.make_async_copy(
                cbuf.at[s], out_hbm.at[li, pl.ds(b0, CH)],
                sem_w.at[s]).start()

        def wait_write(s):
            pltpu.make_async_copy(
                cbuf.at[s], out_hbm.at[0, pl.ds(0, CH)], sem_w.at[s]).wait()

        def step(j, s, drain, prefetch):
            if drain:
                wait_write((s - LAG) % NBUF)
            if prefetch:
                start_load(j + PF, (s + PF) % NBUF)
            wait_load(j, s)
            start_write(j, s)

        for j in range(PF):
            start_load(j, j)

        w0 = NBUF
        w1 = w0 + ((nch - PF - w0) // NBUF) * NBUF
        for j in range(w0):
            step(j, j % NBUF, drain=j >= LAG, prefetch=j + PF < nch)

        def loop_body(k, carry):
            for bb in range(NBUF):
                step(w0 + k * NBUF + bb, bb, drain=True, prefetch=True)
            return carry

        lax.fori_loop(0, (w1 - w0) // NBUF, loop_body, 0)

        for j in range(w1, nch):
            step(j, j % NBUF, drain=j >= LAG, prefetch=j + PF < nch)
        for j in range(nch - LAG, nch):
            wait_write(j % NBUF)

    return body


def kernel(indices, other_features, table):
    b, l = indices.shape
    idx_t = jnp.transpose(indices).astype(jnp.int32)
    other_t = jnp.transpose(other_features, (1, 0, 2))
    table_pad = jnp.concatenate(
        [table, jnp.zeros_like(table)], axis=1)
    out_t = _sc_embed_concat(b, l)(idx_t, other_t, table_pad)
    return jnp.transpose(out_t, (1, 0, 2))


# speedup vs baseline: 4.7431x; 1.1216x over previous
import functools

import jax
import jax.numpy as jnp
from jax import lax
from jax.experimental import pallas as pl
from jax.experimental.pallas import tpu as pltpu
from jax.experimental.pallas import tpu_sc as plsc

D = 64
CH = 64
NBUF = 6
PF = 3
LAG = NBUF - PF
NC = 2
NS = 16
NW = NC * NS
RU = 4


def _sc_embed_concat(b, l):
    slab = b // NW
    nch = l * (slab // CH)
    mesh = plsc.VectorSubcoreMesh(core_axis_name="c", subcore_axis_name="s")

    @functools.partial(
        pl.kernel,
        out_type=jax.ShapeDtypeStruct((l, b, 2 * D), jnp.float32),
        mesh=mesh,
        scratch_types=[
            pltpu.VMEM((l, slab), jnp.int32),
            pltpu.VMEM((NBUF, CH, 2 * D), jnp.float32),
            pltpu.VMEM((NBUF, CH, D), jnp.float32),
            pltpu.SemaphoreType.DMA,
            pltpu.SemaphoreType.DMA((NBUF,)),
            pltpu.SemaphoreType.DMA((NBUF,)),
            pltpu.SemaphoreType.DMA((NBUF,)),
        ],
    )
    def body(idx_hbm, other_hbm, table_hbm, out_hbm,
             idx_v, cbuf, obuf, sem_i, sem_g, sem_o, sem_w):
        wid = lax.axis_index("s") * NC + lax.axis_index("c")
        base = pl.multiple_of(wid * slab, slab)
        pltpu.async_copy(
            idx_hbm.at[:, pl.ds(base, slab)], idx_v, sem_i).wait()

        nh = slab // CH

        def addr(j):
            return j // nh, (j % nh) * CH

        def start_load(j, s):
            li, off = addr(j)
            b0 = pl.multiple_of(base + off, CH)
            pltpu.make_async_copy(
                table_hbm.at[idx_v.at[li, pl.ds(off, CH)]], cbuf.at[s],
                sem_g.at[s]).start()
            pltpu.make_async_copy(
                other_hbm.at[li, pl.ds(b0, CH)], obuf.at[s],
                sem_o.at[s]).start()

        def wait_load(j, s):
            li, off = addr(j)
            b0 = pl.multiple_of(base + off, CH)
            pltpu.make_async_copy(
                table_hbm.at[idx_v.at[li, pl.ds(off, CH)]], cbuf.at[s],
                sem_g.at[s]).wait()
            pltpu.make_async_copy(
                other_hbm.at[li, pl.ds(b0, CH)], obuf.at[s],
                sem_o.at[s]).wait()

        def start_write(j, s):
            def rows(i, carry):
                for k in range(RU):
                    r = i * RU + k
                    for c in range(D // 16):
                        cbuf[s, r, pl.ds(D + c * 16, 16)] = (
                            obuf[s, r, pl.ds(c * 16, 16)])
                return carry

            lax.fori_loop(0, CH // RU, rows, 0)
            li, off = addr(j)
            b0 = pl.multiple_of(base + off, CH)
            pltpu.make_async_copy(
                cbuf.at[s], out_hbm.at[li, pl.ds(b0, CH)],
                sem_w.at[s]).start()

        def wait_write(s):
            pltpu.make_async_copy(
                cbuf.at[s], out_hbm.at[0, pl.ds(0, CH)], sem_w.at[s]).wait()

        def step(j, s, drain, prefetch):
            if drain:
                wait_write((s - LAG) % NBUF)
            if prefetch:
                start_load(j + PF, (s + PF) % NBUF)
            wait_load(j, s)
            start_write(j, s)

        for j in range(PF):
            start_load(j, j)

        w0 = NBUF
        w1 = w0 + ((nch - PF - w0) // NBUF) * NBUF
        for j in range(w0):
            step(j, j % NBUF, drain=j >= LAG, prefetch=j + PF < nch)

        def loop_body(k, carry):
            for bb in range(NBUF):
                step(w0 + k * NBUF + bb, bb, drain=True, prefetch=True)
            return carry

        lax.fori_loop(0, (w1 - w0) // NBUF, loop_body, 0)

        for j in range(w1, nch):
            step(j, j % NBUF, drain=j >= LAG, prefetch=j + PF < nch)
        for j in range(nch - LAG, nch):
            wait_write(j % NBUF)

    return body


def _tc_pad_table(table):
    v = table.shape[0]
    bk = 2048
    tt = jnp.transpose(table)

    def body(tt_ref, out_ref):
        out_ref[:, 0:D] = jnp.transpose(tt_ref[...])
        out_ref[:, D:] = jnp.zeros((bk, D), jnp.float32)

    return pl.pallas_call(
        body,
        grid=(pl.cdiv(v, bk),),
        in_specs=[pl.BlockSpec((D, bk), lambda i: (0, i))],
        out_specs=pl.BlockSpec((bk, 2 * D), lambda i: (i, 0)),
        out_shape=jax.ShapeDtypeStruct((v, 2 * D), jnp.float32),
    )(tt)


def kernel(indices, other_features, table):
    b, l = indices.shape
    idx_t = jnp.transpose(indices).astype(jnp.int32)
    other_t = jnp.transpose(other_features, (1, 0, 2))
    table_pad = _tc_pad_table(table)
    out_t = _sc_embed_concat(b, l)(idx_t, other_t, table_pad)
    return jnp.transpose(out_t, (1, 0, 2))
